# Initial kernel scaffold; baseline (speedup 1.0000x reference)
import jax
import jax.numpy as jnp
from jax.experimental import pallas as pl


def kernel(h, probs, log_sigma_l):
    raise NotImplementedError("write your pallas kernel here")


# speedup vs baseline: 1.6216x; 1.6216x over previous
import functools

import jax
import jax.numpy as jnp
from jax.experimental import pallas as pl
from jax.experimental.pallas import tpu as pltpu


def _protos_body(hc_ref, probs_ref, protos_ref, psq_ref):
    i = pl.program_id(0)
    nb = pl.num_programs(0)
    pb = probs_ref[...]
    hb = hc_ref[...]
    part = jax.lax.dot_general(
        pb, hb, (((0,), (0,)), ((), ())),
        preferred_element_type=jnp.float32,
        precision=jax.lax.Precision.HIGHEST)
    ssum = jnp.sum(pb, axis=0)[None, :]

    @pl.when(i == 0)
    def _():
        protos_ref[...] = part
        psq_ref[...] = ssum

    @pl.when(i > 0)
    def _():
        protos_ref[...] += part
        psq_ref[...] += ssum

    @pl.when(i == nb - 1)
    def _():
        s = psq_ref[0, :]
        s = jnp.where(s == 0.0, 1.0, s)
        pr = protos_ref[...] / s[:, None]
        protos_ref[...] = pr
        psq_ref[...] = jnp.sum(pr * pr, axis=1)[None, :]


def _logits_body(ls_ref, hc_ref, protos_ref, psq_ref, out_ref):
    hb = hc_ref[...]
    pr = protos_ref[...]
    cross = jax.lax.dot_general(
        hb, pr, (((1,), (1,)), ((), ())),
        preferred_element_type=jnp.float32,
        precision=jax.lax.Precision.HIGHEST)
    h_sq = jnp.sum(hb * hb, axis=1, keepdims=True)
    scale = -0.5 * jnp.exp(-ls_ref[0])
    out_ref[...] = (h_sq - 2.0 * cross + psq_ref[...]) * scale


@functools.partial(jax.jit, static_argnames=("interpret",))
def _run(h, probs, log_sigma_l, interpret=False):
    B, N, two, D = h.shape
    K = probs.shape[-1]
    D2 = two * D
    hc = h.reshape(N, D2)
    pz = probs.reshape(N, K)

    nb_a = 4
    protos, psq = pl.pallas_call(
        _protos_body,
        grid=(nb_a,),
        in_specs=[
            pl.BlockSpec((N // nb_a, D2), lambda i: (i, 0)),
            pl.BlockSpec((N // nb_a, K), lambda i: (i, 0)),
        ],
        out_specs=[
            pl.BlockSpec((K, D2), lambda i: (0, 0)),
            pl.BlockSpec((1, K), lambda i: (0, 0)),
        ],
        out_shape=[
            jax.ShapeDtypeStruct((K, D2), jnp.float32),
            jax.ShapeDtypeStruct((1, K), jnp.float32),
        ],
        interpret=interpret,
    )(hc, pz)

    nb_b = 4
    out = pl.pallas_call(
        _logits_body,
        grid=(nb_b,),
        in_specs=[
            pl.BlockSpec(memory_space=pltpu.SMEM),
            pl.BlockSpec((N // nb_b, D2), lambda i: (i, 0)),
            pl.BlockSpec((K, D2), lambda i: (0, 0)),
            pl.BlockSpec((1, K), lambda i: (0, 0)),
        ],
        out_specs=pl.BlockSpec((N // nb_b, K), lambda i: (i, 0)),
        out_shape=jax.ShapeDtypeStruct((N, K), jnp.float32),
        interpret=interpret,
    )(log_sigma_l, hc, protos, psq)

    return out.reshape(B, N, K)


def kernel(h, probs, log_sigma_l):
    return _run(h, probs, log_sigma_l)


# speedup vs baseline: 2.8182x; 1.7379x over previous
import functools

import jax
import jax.numpy as jnp
from jax.experimental import pallas as pl
from jax.experimental.pallas import tpu as pltpu


def _protos_body(hc_ref, probs_ref, protos_ref, psq_ref):
    i = pl.program_id(0)
    nb = pl.num_programs(0)
    pb = probs_ref[...]
    hb = hc_ref[...]
    part = jax.lax.dot_general(
        pb.astype(jnp.bfloat16), hb.astype(jnp.bfloat16),
        (((0,), (0,)), ((), ())),
        preferred_element_type=jnp.float32)
    ssum = jnp.sum(pb, axis=0)[None, :]

    @pl.when(i == 0)
    def _():
        protos_ref[...] = part
        psq_ref[...] = ssum

    @pl.when(i > 0)
    def _():
        protos_ref[...] += part
        psq_ref[...] += ssum

    @pl.when(i == nb - 1)
    def _():
        s = psq_ref[0, :]
        s = jnp.where(s == 0.0, 1.0, s)
        pr = protos_ref[...] / s[:, None]
        protos_ref[...] = pr
        psq_ref[...] = jnp.sum(pr * pr, axis=1)[None, :]


def _logits_body(ls_ref, hc_ref, protos_ref, psq_ref, out_ref):
    hb = hc_ref[...]
    pr = protos_ref[...]
    cross = jax.lax.dot_general(
        hb.astype(jnp.bfloat16), pr.astype(jnp.bfloat16),
        (((1,), (1,)), ((), ())),
        preferred_element_type=jnp.float32)
    h_sq = jnp.sum(hb * hb, axis=1, keepdims=True)
    scale = -0.5 * jnp.exp(-ls_ref[0])
    out_ref[...] = (h_sq - 2.0 * cross + psq_ref[...]) * scale


@functools.partial(jax.jit, static_argnames=("interpret",))
def _run(h, probs, log_sigma_l, interpret=False):
    B, N, two, D = h.shape
    K = probs.shape[-1]
    D2 = two * D
    hc = h.reshape(N, D2)
    pz = probs.reshape(N, K)

    nb_a = 4
    protos, psq = pl.pallas_call(
        _protos_body,
        grid=(nb_a,),
        in_specs=[
            pl.BlockSpec((N // nb_a, D2), lambda i: (i, 0)),
            pl.BlockSpec((N // nb_a, K), lambda i: (i, 0)),
        ],
        out_specs=[
            pl.BlockSpec((K, D2), lambda i: (0, 0)),
            pl.BlockSpec((1, K), lambda i: (0, 0)),
        ],
        out_shape=[
            jax.ShapeDtypeStruct((K, D2), jnp.float32),
            jax.ShapeDtypeStruct((1, K), jnp.float32),
        ],
        interpret=interpret,
    )(hc, pz)

    nb_b = 4
    out = pl.pallas_call(
        _logits_body,
        grid=(nb_b,),
        in_specs=[
            pl.BlockSpec(memory_space=pltpu.SMEM),
            pl.BlockSpec((N // nb_b, D2), lambda i: (i, 0)),
            pl.BlockSpec((K, D2), lambda i: (0, 0)),
            pl.BlockSpec((1, K), lambda i: (0, 0)),
        ],
        out_specs=pl.BlockSpec((N // nb_b, K), lambda i: (i, 0)),
        out_shape=jax.ShapeDtypeStruct((N, K), jnp.float32),
        interpret=interpret,
    )(log_sigma_l, hc, protos, psq)

    return out.reshape(B, N, K)


def kernel(h, probs, log_sigma_l):
    return _run(h, probs, log_sigma_l)
